# baseline (speedup 1.0000x reference)
import functools

import jax
import jax.numpy as jnp
from jax import lax
from jax.experimental import pallas as pl
from jax.experimental.pallas import tpu as pltpu
from jax.experimental.pallas import tpu_sc as plsc

N_USERS_K = 20000
N_ITEMS_K = 30000
N_NODES_K = N_USERS_K + N_ITEMS_K
NNZ_K = 800000
D_K = 64
B_K = 4096

NC = 2
NS = 16
L = 16

HALF = N_NODES_K // NC
SUB_ROWS = 1568
PAD_HALF = NS * SUB_ROWS
DUMMY_ROW = PAD_HALF
ACC_ROWS = PAD_HALF + NS
N_PAD = NC * PAD_HALF
PAD_SHIFT = PAD_HALF - HALF

EDGES_PER_SUB = 51200
NNZ_PAD = NS * EDGES_PER_SUB
CHUNK = 1024
SUBCHUNK = 128
N_CHUNKS = EDGES_PER_SUB // CHUNK
COPY_ROWS = 112
N_COPIES = SUB_ROWS // COPY_ROWS

_mesh = plsc.VectorSubcoreMesh(core_axis_name="c", subcore_axis_name="s")
_cparams = pltpu.CompilerParams(use_tc_tiling_on_sc=False)


@functools.partial(
    pl.kernel,
    mesh=_mesh,
    out_type=jax.ShapeDtypeStruct((N_PAD, D_K), jnp.float32),
    compiler_params=_cparams,
    scratch_types=[pltpu.VMEM((COPY_ROWS, D_K), jnp.float32)],
)
def _pad_copy(src, dst, buf_v):
    cid = lax.axis_index("c")
    sid = lax.axis_index("s")
    src0 = cid * HALF + sid * SUB_ROWS
    dst0 = cid * PAD_HALF + sid * SUB_ROWS
    for j in range(N_COPIES):
        s_raw = src0 + j * COPY_ROWS
        s = jnp.minimum(s_raw, N_NODES_K - COPY_ROWS)
        d = dst0 + j * COPY_ROWS - (s_raw - s)
        pltpu.sync_copy(src.at[pl.ds(s, COPY_ROWS)], buf_v)
        pltpu.sync_copy(buf_v, dst.at[pl.ds(d, COPY_ROWS)])


@functools.partial(
    pl.kernel,
    mesh=_mesh,
    out_type=jax.ShapeDtypeStruct((N_PAD, D_K), jnp.float32),
    compiler_params=_cparams,
    scratch_types=[
        pltpu.VMEM((CHUNK,), jnp.int32),
        pltpu.VMEM((CHUNK,), jnp.int32),
        pltpu.VMEM((CHUNK,), jnp.float32),
        [pltpu.VMEM((SUBCHUNK,), jnp.int32)] * 2,
        [pltpu.VMEM((SUBCHUNK,), jnp.int32)] * 2,
        [pltpu.VMEM((SUBCHUNK, D_K), jnp.float32)] * 2,
        pltpu.VMEM((COPY_ROWS, D_K), jnp.float32),
        pltpu.VMEM_SHARED((ACC_ROWS, D_K), jnp.float32),
        [pltpu.SemaphoreType.DMA] * 2,
        [pltpu.SemaphoreType.DMA] * 2,
    ],
)
def _spmm(e_in, col_hbm, row_hbm, val_hbm, e_out,
          col_v, row_v, val_v, gidx_v, sidx_v, rows_v, buf_v, acc, gsem, ssem):
    cid = lax.axis_index("c")
    sid = lax.axis_index("s")
    row_base = cid * HALF

    def zero_body(i, _):
        for k in range(D_K // L):
            buf_v[i, pl.ds(k * L, L)] = jnp.zeros((L,), jnp.float32)
        return 0
    lax.fori_loop(0, COPY_ROWS, zero_body, 0)
    stripe0 = sid * SUB_ROWS
    for j in range(N_COPIES):
        pltpu.sync_copy(buf_v, acc.at[pl.ds(stripe0 + j * COPY_ROWS, COPY_ROWS)])
    plsc.subcore_barrier()

    n_subs = CHUNK // SUBCHUNK

    def stage_idx(sub, slot):
        def idx_body(i, _):
            off = pl.ds(sub * SUBCHUNK + i * L, L)
            r = row_v[off]
            cc = col_v[off]
            rl = r - row_base
            inb = (rl >= 0) & (rl < HALF)
            sidx_v[slot][pl.ds(i * L, L)] = jnp.full((L,), DUMMY_ROW, jnp.int32)
            cc = cc + jnp.where(cc >= HALF, jnp.int32(PAD_SHIFT), jnp.int32(0))
            gidx_v[slot][pl.ds(i * L, L)] = cc
            return 0
        lax.fori_loop(0, SUBCHUNK // L, idx_body, 0)

    def mul_rows(sub, slot):
        def mul_body(g, _):
            vv = val_v[pl.ds(sub * SUBCHUNK + g * L, L)]
            for j in range(L):
                e = g * L + j
                v = vv[j]
                for k in range(D_K // L):
                    sl = pl.ds(k * L, L)
                    rows_v[slot][e, sl] = rows_v[slot][e, sl] * v
            return 0
        lax.fori_loop(0, SUBCHUNK // L, mul_body, 0)

    def chunk_body(ch, _):
        ebase = sid * EDGES_PER_SUB + ch * CHUNK
        pltpu.sync_copy(col_hbm.at[pl.ds(ebase, CHUNK)], col_v)
        pltpu.sync_copy(row_hbm.at[pl.ds(ebase, CHUNK)], row_v)
        pltpu.sync_copy(val_hbm.at[pl.ds(ebase, CHUNK)], val_v)
        stage_idx(0, 0)
        pltpu.async_copy(e_in.at[gidx_v[0]], rows_v[0], gsem[0])
        for sub in range(n_subs):
            cur, nxt = sub % 2, (sub + 1) % 2
            if sub + 1 < n_subs:
                if sub > 0:
                    pltpu.make_async_copy(
                        rows_v[nxt], acc.at[sidx_v[nxt]], ssem[nxt]).wait()
                stage_idx(sub + 1, nxt)
                pltpu.async_copy(e_in.at[gidx_v[nxt]], rows_v[nxt], gsem[nxt])
            pltpu.make_async_copy(e_in.at[gidx_v[cur]], rows_v[cur],
                                  gsem[cur]).wait()
            mul_rows(sub, cur)
            pltpu.async_copy(rows_v[cur], acc.at[sidx_v[cur]], ssem[cur],
                             add=True)
        for slot in range(2):
            pltpu.make_async_copy(rows_v[slot], acc.at[sidx_v[slot]],
                                  ssem[slot]).wait()
        return 0
    lax.fori_loop(0, N_CHUNKS, chunk_body, 0)
    plsc.subcore_barrier()

    out0 = cid * PAD_HALF + sid * SUB_ROWS
    for j in range(N_COPIES):
        pltpu.sync_copy(acc.at[pl.ds(stripe0 + j * COPY_ROWS, COPY_ROWS)], buf_v)
        pltpu.sync_copy(buf_v, e_out.at[pl.ds(out0 + j * COPY_ROWS, COPY_ROWS)])


_B_PER_W = B_K // (NC * NS)

_out_sds = jax.ShapeDtypeStruct((B_K, D_K), jnp.float32)


@functools.partial(
    pl.kernel,
    mesh=_mesh,
    out_type=(_out_sds,) * 6,
    compiler_params=_cparams,
    scratch_types=[
        pltpu.VMEM((_B_PER_W,), jnp.int32),
        pltpu.VMEM((_B_PER_W,), jnp.int32),
        pltpu.VMEM((_B_PER_W,), jnp.int32),
        pltpu.VMEM((_B_PER_W, D_K), jnp.float32),
        pltpu.VMEM((_B_PER_W, D_K), jnp.float32),
        pltpu.SemaphoreType.DMA,
    ],
)
def _gather_mean(e0, t1, t2, t3, users_hbm, pos_hbm, neg_hbm,
                 u_emb, p_emb, n_emb, u_emb0, p_emb0, n_emb0,
                 stage_v, nid0_v, nidp_v, sum_v, gt_v, sem):
    cid = lax.axis_index("c")
    sid = lax.axis_index("s")
    wid = sid * NC + cid
    tb = wid * _B_PER_W

    for idx_hbm, emb_out, emb0_out, base in (
            (users_hbm, u_emb, u_emb0, 0),
            (pos_hbm, p_emb, p_emb0, N_USERS_K),
            (neg_hbm, n_emb, n_emb0, N_USERS_K)):
        pltpu.sync_copy(idx_hbm.at[pl.ds(tb, _B_PER_W)], stage_v)

        def idx_body(i, _):
            x = stage_v[pl.ds(i * L, L)] + base
            nid0_v[pl.ds(i * L, L)] = x
            nidp_v[pl.ds(i * L, L)] = x + jnp.where(
                x >= HALF, jnp.int32(PAD_SHIFT), jnp.int32(0))
            return 0
        lax.fori_loop(0, _B_PER_W // L, idx_body, 0)

        pltpu.async_copy(e0.at[nid0_v], sum_v, sem).wait()
        pltpu.sync_copy(sum_v, emb0_out.at[pl.ds(tb, _B_PER_W)])

        for t in (t1, t2, t3):
            pltpu.async_copy(t.at[nidp_v], gt_v, sem).wait()

            def add_body(e, _):
                for k in range(D_K // L):
                    sl = pl.ds(k * L, L)
                    sum_v[e, sl] = sum_v[e, sl] + gt_v[e, sl]
                return 0
            lax.fori_loop(0, _B_PER_W, add_body, 0, unroll=4)

        def scale_body(e, _):
            for k in range(D_K // L):
                sl = pl.ds(k * L, L)
                sum_v[e, sl] = sum_v[e, sl] * jnp.float32(0.25)
            return 0
        lax.fori_loop(0, _B_PER_W, scale_body, 0, unroll=4)
        pltpu.sync_copy(sum_v, emb_out.at[pl.ds(tb, _B_PER_W)])


def kernel(E0, adj_values, adj_indices, users, pos_items, neg_items):
    row = adj_indices[0].astype(jnp.int32)
    col = adj_indices[1].astype(jnp.int32)
    pad = NNZ_PAD - NNZ_K
    col_p = jnp.concatenate([col, jnp.zeros((pad,), jnp.int32)])
    row_p = jnp.concatenate([row, jnp.zeros((pad,), jnp.int32)])
    val_p = jnp.concatenate([adj_values, jnp.zeros((pad,), jnp.float32)])

    e0p = _pad_copy(E0)
    t1 = _spmm(e0p, col_p, row_p, val_p)
    t2 = _spmm(t1, col_p, row_p, val_p)
    t3 = _spmm(t2, col_p, row_p, val_p)

    return _gather_mean(E0, t1, t2, t3,
                        users.astype(jnp.int32),
                        pos_items.astype(jnp.int32),
                        neg_items.astype(jnp.int32))


# speedup vs baseline: 1.2037x; 1.2037x over previous
import functools

import jax
import jax.numpy as jnp
from jax import lax
from jax.experimental import pallas as pl
from jax.experimental.pallas import tpu as pltpu
from jax.experimental.pallas import tpu_sc as plsc

N_USERS_K = 20000
N_ITEMS_K = 30000
N_NODES_K = N_USERS_K + N_ITEMS_K
NNZ_K = 800000
D_K = 64
B_K = 4096

NC = 2
NS = 16
L = 16

HALF = N_NODES_K // NC
SUB_ROWS = 1568
PAD_HALF = NS * SUB_ROWS
DUMMY_ROW = PAD_HALF
ACC_ROWS = PAD_HALF + NS
N_PAD = NC * PAD_HALF
PAD_SHIFT = PAD_HALF - HALF

EDGES_PER_SUB = 51200
NNZ_PAD = NS * EDGES_PER_SUB
CHUNK = 2048
SUBCHUNK = 128
N_CHUNKS = EDGES_PER_SUB // CHUNK
COPY_ROWS = 112
N_COPIES = SUB_ROWS // COPY_ROWS

_mesh = plsc.VectorSubcoreMesh(core_axis_name="c", subcore_axis_name="s")
_cparams = pltpu.CompilerParams(use_tc_tiling_on_sc=False)


@functools.partial(
    pl.kernel,
    mesh=_mesh,
    out_type=jax.ShapeDtypeStruct((N_PAD, D_K), jnp.float32),
    compiler_params=_cparams,
    scratch_types=[pltpu.VMEM((COPY_ROWS, D_K), jnp.float32)],
)
def _pad_copy(src, dst, buf_v):
    cid = lax.axis_index("c")
    sid = lax.axis_index("s")
    src0 = cid * HALF + sid * SUB_ROWS
    dst0 = cid * PAD_HALF + sid * SUB_ROWS
    for j in range(N_COPIES):
        s_raw = src0 + j * COPY_ROWS
        s = jnp.minimum(s_raw, N_NODES_K - COPY_ROWS)
        d = dst0 + j * COPY_ROWS - (s_raw - s)
        pltpu.sync_copy(src.at[pl.ds(s, COPY_ROWS)], buf_v)
        pltpu.sync_copy(buf_v, dst.at[pl.ds(d, COPY_ROWS)])


@functools.partial(
    pl.kernel,
    mesh=_mesh,
    out_type=jax.ShapeDtypeStruct((N_PAD, D_K), jnp.float32),
    compiler_params=_cparams,
    scratch_types=[
        pltpu.VMEM((CHUNK,), jnp.int32),
        pltpu.VMEM((CHUNK,), jnp.int32),
        pltpu.VMEM((CHUNK,), jnp.float32),
        [pltpu.VMEM((SUBCHUNK,), jnp.int32)] * 2,
        [pltpu.VMEM((SUBCHUNK,), jnp.int32)] * 2,
        [pltpu.VMEM((SUBCHUNK, D_K), jnp.float32)] * 2,
        pltpu.VMEM((COPY_ROWS, D_K), jnp.float32),
        pltpu.VMEM_SHARED((ACC_ROWS, D_K), jnp.float32),
        [pltpu.SemaphoreType.DMA] * 2,
        [pltpu.SemaphoreType.DMA] * 2,
    ],
)
def _spmm(e_in, col_hbm, row_hbm, val_hbm, e_out,
          col_v, row_v, val_v, gidx_v, sidx_v, rows_v, buf_v, acc, gsem, ssem):
    cid = lax.axis_index("c")
    sid = lax.axis_index("s")
    row_base = cid * HALF

    def zero_body(i, _):
        for k in range(D_K // L):
            buf_v[i, pl.ds(k * L, L)] = jnp.zeros((L,), jnp.float32)
        return 0
    lax.fori_loop(0, COPY_ROWS, zero_body, 0)
    stripe0 = sid * SUB_ROWS
    for j in range(N_COPIES):
        pltpu.sync_copy(buf_v, acc.at[pl.ds(stripe0 + j * COPY_ROWS, COPY_ROWS)])
    plsc.subcore_barrier()

    n_subs = CHUNK // SUBCHUNK

    def stage_idx(sub, slot):
        def idx_body(i, _):
            off = pl.ds(sub * SUBCHUNK + i * L, L)
            r = row_v[off]
            cc = col_v[off]
            rl = r - row_base
            inb = (rl >= 0) & (rl < HALF)
            sidx_v[slot][pl.ds(i * L, L)] = jnp.where(
                inb, rl, jnp.full((L,), DUMMY_ROW, jnp.int32))
            cc = cc + jnp.where(cc >= HALF, jnp.int32(PAD_SHIFT), jnp.int32(0))
            gidx_v[slot][pl.ds(i * L, L)] = cc
            return 0
        lax.fori_loop(0, SUBCHUNK // L, idx_body, 0)

    def mul_rows(sub, slot):
        def mul_body(g, _):
            vv = val_v[pl.ds(sub * SUBCHUNK + g * L, L)]
            for j in range(L):
                e = g * L + j
                v = vv[j]
                for k in range(D_K // L):
                    sl = pl.ds(k * L, L)
                    rows_v[slot][e, sl] = rows_v[slot][e, sl] * v
            return 0
        lax.fori_loop(0, SUBCHUNK // L, mul_body, 0)

    def chunk_body(ch, _):
        ebase = sid * EDGES_PER_SUB + ch * CHUNK
        pltpu.sync_copy(col_hbm.at[pl.ds(ebase, CHUNK)], col_v)
        pltpu.sync_copy(row_hbm.at[pl.ds(ebase, CHUNK)], row_v)
        pltpu.sync_copy(val_hbm.at[pl.ds(ebase, CHUNK)], val_v)
        stage_idx(0, 0)
        pltpu.async_copy(e_in.at[gidx_v[0]], rows_v[0], gsem[0])
        for sub in range(n_subs):
            cur, nxt = sub % 2, (sub + 1) % 2
            if sub + 1 < n_subs:
                if sub > 0:
                    pltpu.make_async_copy(
                        rows_v[nxt], acc.at[sidx_v[nxt]], ssem[nxt]).wait()
                stage_idx(sub + 1, nxt)
                pltpu.async_copy(e_in.at[gidx_v[nxt]], rows_v[nxt], gsem[nxt])
            pltpu.make_async_copy(e_in.at[gidx_v[cur]], rows_v[cur],
                                  gsem[cur]).wait()
            mul_rows(sub, cur)
            pltpu.async_copy(rows_v[cur], acc.at[sidx_v[cur]], ssem[cur],
                             add=True)
        for slot in range(2):
            pltpu.make_async_copy(rows_v[slot], acc.at[sidx_v[slot]],
                                  ssem[slot]).wait()
        return 0
    lax.fori_loop(0, N_CHUNKS, chunk_body, 0)
    plsc.subcore_barrier()

    out0 = cid * PAD_HALF + sid * SUB_ROWS
    for j in range(N_COPIES):
        pltpu.sync_copy(acc.at[pl.ds(stripe0 + j * COPY_ROWS, COPY_ROWS)], buf_v)
        pltpu.sync_copy(buf_v, e_out.at[pl.ds(out0 + j * COPY_ROWS, COPY_ROWS)])


_B_PER_W = B_K // (NC * NS)

_out_sds = jax.ShapeDtypeStruct((B_K, D_K), jnp.float32)


@functools.partial(
    pl.kernel,
    mesh=_mesh,
    out_type=(_out_sds,) * 6,
    compiler_params=_cparams,
    scratch_types=[
        pltpu.VMEM((_B_PER_W,), jnp.int32),
        pltpu.VMEM((_B_PER_W,), jnp.int32),
        pltpu.VMEM((_B_PER_W,), jnp.int32),
        pltpu.VMEM((_B_PER_W, D_K), jnp.float32),
        pltpu.VMEM((_B_PER_W, D_K), jnp.float32),
        pltpu.SemaphoreType.DMA,
    ],
)
def _gather_mean(e0, t1, t2, t3, users_hbm, pos_hbm, neg_hbm,
                 u_emb, p_emb, n_emb, u_emb0, p_emb0, n_emb0,
                 stage_v, nid0_v, nidp_v, sum_v, gt_v, sem):
    cid = lax.axis_index("c")
    sid = lax.axis_index("s")
    wid = sid * NC + cid
    tb = wid * _B_PER_W

    for idx_hbm, emb_out, emb0_out, base in (
            (users_hbm, u_emb, u_emb0, 0),
            (pos_hbm, p_emb, p_emb0, N_USERS_K),
            (neg_hbm, n_emb, n_emb0, N_USERS_K)):
        pltpu.sync_copy(idx_hbm.at[pl.ds(tb, _B_PER_W)], stage_v)

        def idx_body(i, _):
            x = stage_v[pl.ds(i * L, L)] + base
            nid0_v[pl.ds(i * L, L)] = x
            nidp_v[pl.ds(i * L, L)] = x + jnp.where(
                x >= HALF, jnp.int32(PAD_SHIFT), jnp.int32(0))
            return 0
        lax.fori_loop(0, _B_PER_W // L, idx_body, 0)

        pltpu.async_copy(e0.at[nid0_v], sum_v, sem).wait()
        pltpu.sync_copy(sum_v, emb0_out.at[pl.ds(tb, _B_PER_W)])

        for t in (t1, t2, t3):
            pltpu.async_copy(t.at[nidp_v], gt_v, sem).wait()

            def add_body(e, _):
                for k in range(D_K // L):
                    sl = pl.ds(k * L, L)
                    sum_v[e, sl] = sum_v[e, sl] + gt_v[e, sl]
                return 0
            lax.fori_loop(0, _B_PER_W, add_body, 0, unroll=4)

        def scale_body(e, _):
            for k in range(D_K // L):
                sl = pl.ds(k * L, L)
                sum_v[e, sl] = sum_v[e, sl] * jnp.float32(0.25)
            return 0
        lax.fori_loop(0, _B_PER_W, scale_body, 0, unroll=4)
        pltpu.sync_copy(sum_v, emb_out.at[pl.ds(tb, _B_PER_W)])


def kernel(E0, adj_values, adj_indices, users, pos_items, neg_items):
    row = adj_indices[0].astype(jnp.int32)
    col = adj_indices[1].astype(jnp.int32)
    pad = NNZ_PAD - NNZ_K
    col_p = jnp.concatenate([col, jnp.zeros((pad,), jnp.int32)])
    row_p = jnp.concatenate([row, jnp.zeros((pad,), jnp.int32)])
    val_p = jnp.concatenate([adj_values, jnp.zeros((pad,), jnp.float32)])

    e0p = _pad_copy(E0)
    t1 = _spmm(e0p, col_p, row_p, val_p)
    t2 = _spmm(t1, col_p, row_p, val_p)
    t3 = _spmm(t2, col_p, row_p, val_p)

    return _gather_mean(E0, t1, t2, t3,
                        users.astype(jnp.int32),
                        pos_items.astype(jnp.int32),
                        neg_items.astype(jnp.int32))
